# baseline (device time: 103798 ns/iter reference)
import jax
import jax.numpy as jnp
from jax import lax
from jax.experimental import pallas as pl
from jax.experimental.pallas import tpu as pltpu

N_DEV = 4


def kernel(x, Wg, Wu, Wd):
    m, _ = x.shape
    d = Wd.shape[1]

    xb = x.astype(jnp.bfloat16)
    Wgb = Wg.astype(jnp.bfloat16)
    Wub = Wu.astype(jnp.bfloat16)
    Wdb = Wd.astype(jnp.bfloat16)

    def body(x_ref, wg_ref, wu_ref, wd_ref, out_ref, comm_ref, send_sems, recv_sems):
        my = lax.axis_index("i")
        left = (my + N_DEV - 1) % N_DEV
        right = (my + 1) % N_DEV

        barrier_sem = pltpu.get_barrier_semaphore()
        for nbr in (left, right):
            pl.semaphore_signal(
                barrier_sem, inc=1,
                device_id=(nbr,), device_id_type=pl.DeviceIdType.MESH,
            )
        pl.semaphore_wait(barrier_sem, 2)

        xv = x_ref[...]
        gate = jnp.dot(xv, wg_ref[...], preferred_element_type=jnp.float32)
        up = jnp.dot(xv, wu_ref[...], preferred_element_type=jnp.float32)
        hact = (gate * (up * jax.nn.sigmoid(up))).astype(jnp.bfloat16)
        partial = jnp.dot(hact, wd_ref[...], preferred_element_type=jnp.float32)

        out_ref[...] = partial
        comm_ref[0] = partial.astype(jnp.bfloat16)

        for h in range(N_DEV - 1):
            send_slot = h % 2
            recv_slot = (h + 1) % 2
            rdma = pltpu.make_async_remote_copy(
                src_ref=comm_ref.at[send_slot],
                dst_ref=comm_ref.at[recv_slot],
                send_sem=send_sems.at[send_slot],
                recv_sem=recv_sems.at[recv_slot],
                device_id=(right,),
                device_id_type=pl.DeviceIdType.MESH,
            )
            rdma.start()
            rdma.wait()
            out_ref[...] = out_ref[...] + comm_ref[recv_slot].astype(jnp.float32)

    return pl.pallas_call(
        body,
        out_shape=jax.ShapeDtypeStruct((m, d), jnp.float32),
        in_specs=[pl.BlockSpec(memory_space=pltpu.VMEM)] * 4,
        out_specs=pl.BlockSpec(memory_space=pltpu.VMEM),
        scratch_shapes=[
            pltpu.VMEM((2, m, d), jnp.bfloat16),
            pltpu.SemaphoreType.DMA((2,)),
            pltpu.SemaphoreType.DMA((2,)),
        ],
        compiler_params=pltpu.CompilerParams(collective_id=0),
    )(xb, Wgb, Wub, Wdb)


# device time: 48488 ns/iter; 2.1407x vs baseline; 2.1407x over previous
import jax
import jax.numpy as jnp
from jax import lax
from jax.experimental import pallas as pl
from jax.experimental.pallas import tpu as pltpu

N_DEV = 4


def kernel(x, Wg, Wu, Wd):
    m, _ = x.shape
    d = Wd.shape[1]
    c = m // N_DEV

    xb = x.astype(jnp.bfloat16)
    Wgb = Wg.astype(jnp.bfloat16)
    Wub = Wu.astype(jnp.bfloat16)
    Wdb = Wd.astype(jnp.bfloat16)

    def body(x_ref, wg_ref, wu_ref, wd_ref, out_ref,
             rs_send, rs_buf,
             rs_send_sems, rs_recv_sems, ag_send_sems, ag_recv_sems):
        my = lax.axis_index("i")

        barrier_sem = pltpu.get_barrier_semaphore()
        for off in (1, 2, 3):
            pl.semaphore_signal(
                barrier_sem, inc=1,
                device_id=((my + off) % N_DEV,),
                device_id_type=pl.DeviceIdType.MESH,
            )
        pl.semaphore_wait(barrier_sem, 3)

        wd = wd_ref[...]

        def chunk_partial(j):
            xj = x_ref[pl.ds(j * c, c), :]
            gj = jnp.dot(xj, wg_ref[...], preferred_element_type=jnp.float32)
            uj = jnp.dot(xj, wu_ref[...], preferred_element_type=jnp.float32)
            hj = (gj * (uj * jax.nn.sigmoid(uj))).astype(jnp.bfloat16)
            return jnp.dot(hj, wd, preferred_element_type=jnp.float32)

        rs_rdmas = []
        for off in (1, 2, 3):
            j = (my + off) % N_DEV
            rs_send[off - 1] = chunk_partial(j).astype(jnp.bfloat16)
            rdma = pltpu.make_async_remote_copy(
                src_ref=rs_send.at[off - 1],
                dst_ref=rs_buf.at[off - 1],
                send_sem=rs_send_sems.at[off - 1],
                recv_sem=rs_recv_sems.at[off - 1],
                device_id=(j,),
                device_id_type=pl.DeviceIdType.MESH,
            )
            rdma.start()
            rs_rdmas.append(rdma)

        acc = chunk_partial(my)

        for k in range(3):
            rs_rdmas[k].wait_recv()
            acc = acc + rs_buf[k].astype(jnp.float32)
        out_ref[pl.ds(my * c, c), :] = acc.astype(jnp.bfloat16)

        ag_rdmas = []
        for off in (1, 2, 3):
            j = (my + off) % N_DEV
            rdma = pltpu.make_async_remote_copy(
                src_ref=out_ref.at[pl.ds(my * c, c), :],
                dst_ref=out_ref.at[pl.ds(my * c, c), :],
                send_sem=ag_send_sems.at[off - 1],
                recv_sem=ag_recv_sems.at[off - 1],
                device_id=(j,),
                device_id_type=pl.DeviceIdType.MESH,
            )
            rdma.start()
            ag_rdmas.append(rdma)

        for r in ag_rdmas:
            r.wait_recv()
        for r in rs_rdmas + ag_rdmas:
            r.wait_send()

    return pl.pallas_call(
        body,
        out_shape=jax.ShapeDtypeStruct((m, d), jnp.bfloat16),
        in_specs=[pl.BlockSpec(memory_space=pltpu.VMEM)] * 4,
        out_specs=pl.BlockSpec(memory_space=pltpu.VMEM),
        scratch_shapes=[
            pltpu.VMEM((3, c, d), jnp.bfloat16),
            pltpu.VMEM((3, c, d), jnp.bfloat16),
            pltpu.SemaphoreType.DMA((3,)),
            pltpu.SemaphoreType.DMA((3,)),
            pltpu.SemaphoreType.DMA((3,)),
            pltpu.SemaphoreType.DMA((3,)),
        ],
        compiler_params=pltpu.CompilerParams(collective_id=0),
    )(xb, Wgb, Wub, Wdb)
